# baseline (speedup 1.0000x reference)
import functools

import jax
import jax.numpy as jnp
from jax import lax
from jax.experimental import pallas as pl
from jax.experimental.pallas import tpu as pltpu
from jax.experimental.pallas import tpu_sc as plsc

VOCAB = 1000000
DIM = 32
BATCH_ = 16384
HIST = 50

NC = 2
NS = 16
NW = NC * NS
B_PER_W = BATCH_ // NW
CHUNK = 32
ROWS = CHUNK * HIST
ROUNDS = B_PER_W // CHUNK

VC = 1024
NCH = VOCAB // VC
TAIL0 = NCH * VC
TAILA = 512
TAILB = VOCAB - TAIL0 - TAILA
PAIRS = 15


def _untile_kernel(tt_hbm, tail_hbm, out_hbm, blk0, blk1, outblk, sem0, sem1):
    wid = lax.axis_index("s") * NC + lax.axis_index("c")
    iota_lo = lax.iota(jnp.int32, 16)
    iota_hi = iota_lo + 16
    blk_b = (blk0, blk1)
    sems = (sem0, sem1)

    def chunk_id(j):
        return wid + NW * j

    def start(j, p):
        v0 = chunk_id(j) * VC
        pltpu.async_copy(tt_hbm.at[:, pl.ds(v0, VC)], blk_b[p], sems[p])

    def process(j, p, width):
        blk = blk_b[p]

        def row_body(v, c):
            col = jnp.full((16,), v, jnp.int32)
            g0 = plsc.load_gather(blk, [iota_lo, col])
            g1 = plsc.load_gather(blk, [iota_hi, col])
            outblk[pl.ds(v * DIM, 16)] = g0
            outblk[pl.ds(v * DIM + 16, 16)] = g1
            return c

        lax.fori_loop(0, width, row_body, 0)
        pltpu.sync_copy(outblk.at[pl.ds(0, width * DIM)],
                        out_hbm.at[pl.ds(chunk_id(j) * VC * DIM, width * DIM)])

    start(0, 0)

    def pair_body(jj, carry):
        for p in (0, 1):
            j = jj * 2 + p
            nxt = (p + 1) % 2

            @pl.when(jnp.logical_or(j + 1 < 2 * PAIRS,
                                    jnp.logical_and(j + 1 == 2 * PAIRS,
                                                    wid < 16)))
            def _():
                start(j + 1, nxt)

            pltpu.make_async_copy(
                tt_hbm.at[:, pl.ds(chunk_id(j) * VC, VC)], blk_b[p],
                sems[p]).wait()
            process(j, p, VC)
        return carry

    lax.fori_loop(0, PAIRS, pair_body, 0)

    @pl.when(wid < 16)
    def _():
        j = 2 * PAIRS
        pltpu.make_async_copy(
            tt_hbm.at[:, pl.ds(chunk_id(j) * VC, VC)], blk_b[0],
            sems[0]).wait()
        process(j, 0, VC)

    @pl.when(wid == 16)
    def _():
        pltpu.sync_copy(tt_hbm.at[:, pl.ds(TAIL0, TAILA)],
                        blk_b[1].at[:, pl.ds(0, TAILA)])

        def row_body(v, c):
            col = jnp.full((16,), v, jnp.int32)
            g0 = plsc.load_gather(blk_b[1], [iota_lo, col])
            g1 = plsc.load_gather(blk_b[1], [iota_hi, col])
            outblk[pl.ds(v * DIM, 16)] = g0
            outblk[pl.ds(v * DIM + 16, 16)] = g1
            return c

        lax.fori_loop(0, TAILA, row_body, 0)
        pltpu.sync_copy(outblk.at[pl.ds(0, TAILA * DIM)],
                        out_hbm.at[pl.ds(TAIL0 * DIM, TAILA * DIM)])

    @pl.when(wid == 17)
    def _():
        pltpu.sync_copy(tail_hbm, outblk.at[pl.ds(0, TAILB * DIM)])
        pltpu.sync_copy(outblk.at[pl.ds(0, TAILB * DIM)],
                        out_hbm.at[pl.ds((TAIL0 + TAILA) * DIM, TAILB * DIM)])


def _sc_kernel(table_hbm, idxt_hbm, outt_hbm,
               idx2d0, idx2d1, idxf0, idxf1, rows0, rows1, outt_v,
               sem0, sem1):
    wid = lax.axis_index("s") * NC + lax.axis_index("c")
    base_b0 = wid * B_PER_W
    zero = jnp.zeros((16,), jnp.float32)
    inv = jnp.float32(1.0 / HIST)
    iota16 = lax.iota(jnp.int32, 16)
    idx2d_b = (idx2d0, idx2d1)
    idxf_b = (idxf0, idxf1)
    rows_b = (rows0, rows1)
    sems = (sem0, sem1)

    def start(r, p):
        idx2d = idx2d_b[p]
        idxf = idxf_b[p]
        pltpu.sync_copy(idxt_hbm.at[:, pl.ds(base_b0 + r * CHUNK, CHUNK)],
                        idx2d)

        def flat_body(h, c):
            base = h * CHUNK
            idxf[pl.ds(base, 16)] = idx2d[h, pl.ds(0, 16)]
            idxf[pl.ds(base + 16, 16)] = idx2d[h, pl.ds(16, 16)]
            return c

        lax.fori_loop(0, HIST, flat_body, 0)
        pltpu.async_copy(table_hbm.at[idxf], rows_b[p], sems[p])

    def process(r, p):
        rows_v = rows_b[p]

        def batch_body(b, c):
            a0 = zero
            a1 = zero
            c0 = zero
            c1 = zero
            for h in range(0, HIST, 2):
                a0 = a0 + rows_v[h * CHUNK + b, pl.ds(0, 16)]
                a1 = a1 + rows_v[h * CHUNK + b, pl.ds(16, 16)]
                c0 = c0 + rows_v[(h + 1) * CHUNK + b, pl.ds(0, 16)]
                c1 = c1 + rows_v[(h + 1) * CHUNK + b, pl.ds(16, 16)]
            bvec = jnp.full((16,), b, jnp.int32)
            plsc.store_scatter(outt_v, [iota16, bvec], (a0 + c0) * inv)
            plsc.store_scatter(outt_v, [iota16 + 16, bvec], (a1 + c1) * inv)
            return c

        lax.fori_loop(0, CHUNK, batch_body, 0)
        pltpu.sync_copy(outt_v,
                        outt_hbm.at[:, pl.ds(base_b0 + r * CHUNK, CHUNK)])

    start(0, 0)

    def outer(rr, carry):
        for p in (0, 1):
            r = rr * 2 + p
            nxt = (p + 1) % 2

            @pl.when(r + 1 < ROUNDS)
            def _():
                start(r + 1, nxt)

            pltpu.make_async_copy(table_hbm.at[idxf_b[p]], rows_b[p],
                                  sems[p]).wait()
            process(r, p)
        return carry

    lax.fori_loop(0, ROUNDS // 2, outer, 0)


@jax.jit
def _pooled_lookup(emb_table_t, tail_lin):
    mesh = plsc.VectorSubcoreMesh(core_axis_name="c", subcore_axis_name="s")
    untile = functools.partial(
        pl.kernel,
        mesh=mesh,
        out_type=jax.ShapeDtypeStruct((VOCAB * DIM,), jnp.float32),
        scratch_types=[
            pltpu.VMEM((DIM, VC), jnp.float32),
            pltpu.VMEM((DIM, VC), jnp.float32),
            pltpu.VMEM((VC * DIM,), jnp.float32),
            pltpu.SemaphoreType.DMA,
            pltpu.SemaphoreType.DMA,
        ],
        compiler_params=pltpu.CompilerParams(use_tc_tiling_on_sc=True,
                                             needs_layout_passes=False),
    )(_untile_kernel)
    return untile(emb_table_t, tail_lin)


@jax.jit
def _lookup(table_lin, idx_t):
    mesh = plsc.VectorSubcoreMesh(core_axis_name="c", subcore_axis_name="s")
    f = functools.partial(
        pl.kernel,
        mesh=mesh,
        out_type=jax.ShapeDtypeStruct((DIM, BATCH_), jnp.float32),
        scratch_types=[
            pltpu.VMEM((HIST, CHUNK), jnp.int32),
            pltpu.VMEM((HIST, CHUNK), jnp.int32),
            pltpu.VMEM((ROWS,), jnp.int32),
            pltpu.VMEM((ROWS,), jnp.int32),
            pltpu.VMEM((ROWS, DIM), jnp.float32),
            pltpu.VMEM((ROWS, DIM), jnp.float32),
            pltpu.VMEM((DIM, CHUNK), jnp.float32),
            pltpu.SemaphoreType.DMA,
            pltpu.SemaphoreType.DMA,
        ],
        compiler_params=pltpu.CompilerParams(use_tc_tiling_on_sc=False,
                                             needs_layout_passes=False),
    )(_sc_kernel)
    return f(table_lin, idx_t)


def kernel(emb_table, inputs):
    tail_lin = emb_table[VOCAB - 64:].reshape(64 * DIM)
    table_flat = _pooled_lookup(emb_table.T, tail_lin)
    table_lin = table_flat.reshape(VOCAB, DIM)
    out_t = _lookup(table_lin, inputs.T)
    return out_t.T


# speedup vs baseline: 1.0698x; 1.0698x over previous
import functools

import jax
import jax.numpy as jnp
from jax import lax
from jax.experimental import pallas as pl
from jax.experimental.pallas import tpu as pltpu
from jax.experimental.pallas import tpu_sc as plsc

VOCAB = 1000000
DIM = 32
BATCH_ = 16384
HIST = 50

NC = 2
NS = 16
NW = NC * NS
B_PER_W = BATCH_ // NW
CHUNK = 32
ROWS = CHUNK * HIST
ROUNDS = B_PER_W // CHUNK

VC = 1024
NCH = VOCAB // VC
TAIL0 = NCH * VC
TAILA = 512
TAILB = VOCAB - TAIL0 - TAILA
PAIRS = 15


def _untile_kernel(tt_hbm, tail_hbm, out_hbm, blk0, blk1, outblk, sem0, sem1):
    wid = lax.axis_index("s") * NC + lax.axis_index("c")
    iotam32 = lax.iota(jnp.int32, 16) * DIM
    blk_b = (blk0, blk1)
    sems = (sem0, sem1)

    def chunk_id(j):
        return wid + NW * j

    def start(j, p):
        v0 = chunk_id(j) * VC
        pltpu.async_copy(tt_hbm.at[:, pl.ds(v0, VC)], blk_b[p], sems[p])

    def transpose_block(blk, width):
        def g_body(g, c):
            base = g * (16 * DIM)
            for d in range(DIM):
                vec = blk[d, pl.ds(g * 16, 16)]
                plsc.store_scatter(outblk, [iotam32 + (base + d)], vec)
            return c

        lax.fori_loop(0, width // 16, g_body, 0)

    def process(j, p, width):
        transpose_block(blk_b[p], width)
        pltpu.sync_copy(outblk.at[pl.ds(0, width * DIM)],
                        out_hbm.at[pl.ds(chunk_id(j) * VC * DIM, width * DIM)])

    start(0, 0)

    def pair_body(jj, carry):
        for p in (0, 1):
            j = jj * 2 + p
            nxt = (p + 1) % 2

            @pl.when(jnp.logical_or(j + 1 < 2 * PAIRS,
                                    jnp.logical_and(j + 1 == 2 * PAIRS,
                                                    wid < 16)))
            def _():
                start(j + 1, nxt)

            pltpu.make_async_copy(
                tt_hbm.at[:, pl.ds(chunk_id(j) * VC, VC)], blk_b[p],
                sems[p]).wait()
            process(j, p, VC)
        return carry

    lax.fori_loop(0, PAIRS, pair_body, 0)

    @pl.when(wid < 16)
    def _():
        j = 2 * PAIRS
        pltpu.make_async_copy(
            tt_hbm.at[:, pl.ds(chunk_id(j) * VC, VC)], blk_b[0],
            sems[0]).wait()
        process(j, 0, VC)

    @pl.when(wid == 16)
    def _():
        pltpu.sync_copy(tt_hbm.at[:, pl.ds(TAIL0, TAILA)],
                        blk_b[1].at[:, pl.ds(0, TAILA)])
        transpose_block(blk_b[1], TAILA)
        pltpu.sync_copy(outblk.at[pl.ds(0, TAILA * DIM)],
                        out_hbm.at[pl.ds(TAIL0 * DIM, TAILA * DIM)])

    @pl.when(wid == 17)
    def _():
        pltpu.sync_copy(tail_hbm, outblk.at[pl.ds(0, TAILB * DIM)])
        pltpu.sync_copy(outblk.at[pl.ds(0, TAILB * DIM)],
                        out_hbm.at[pl.ds((TAIL0 + TAILA) * DIM, TAILB * DIM)])


def _sc_kernel(table_hbm, idxt_hbm, outt_hbm,
               idx2d0, idx2d1, idxf0, idxf1, rows0, rows1, outt_v,
               sem0, sem1):
    wid = lax.axis_index("s") * NC + lax.axis_index("c")
    base_b0 = wid * B_PER_W
    zero = jnp.zeros((16,), jnp.float32)
    inv = jnp.float32(1.0 / HIST)
    iota16 = lax.iota(jnp.int32, 16)
    idx2d_b = (idx2d0, idx2d1)
    idxf_b = (idxf0, idxf1)
    rows_b = (rows0, rows1)
    sems = (sem0, sem1)

    def start(r, p):
        idx2d = idx2d_b[p]
        idxf = idxf_b[p]
        pltpu.sync_copy(idxt_hbm.at[:, pl.ds(base_b0 + r * CHUNK, CHUNK)],
                        idx2d)

        def flat_body(h, c):
            base = h * CHUNK
            idxf[pl.ds(base, 16)] = idx2d[h, pl.ds(0, 16)]
            idxf[pl.ds(base + 16, 16)] = idx2d[h, pl.ds(16, 16)]
            return c

        lax.fori_loop(0, HIST, flat_body, 0)
        pltpu.async_copy(table_hbm.at[idxf], rows_b[p], sems[p])

    def process(r, p):
        rows_v = rows_b[p]

        def batch_body(b, c):
            a0 = zero
            a1 = zero
            c0 = zero
            c1 = zero
            for h in range(0, HIST, 2):
                a0 = a0 + rows_v[h * CHUNK + b, pl.ds(0, 16)]
                a1 = a1 + rows_v[h * CHUNK + b, pl.ds(16, 16)]
                c0 = c0 + rows_v[(h + 1) * CHUNK + b, pl.ds(0, 16)]
                c1 = c1 + rows_v[(h + 1) * CHUNK + b, pl.ds(16, 16)]
            bvec = jnp.full((16,), b, jnp.int32)
            plsc.store_scatter(outt_v, [iota16, bvec], (a0 + c0) * inv)
            plsc.store_scatter(outt_v, [iota16 + 16, bvec], (a1 + c1) * inv)
            return c

        lax.fori_loop(0, CHUNK, batch_body, 0)
        pltpu.sync_copy(outt_v,
                        outt_hbm.at[:, pl.ds(base_b0 + r * CHUNK, CHUNK)])

    start(0, 0)

    def outer(rr, carry):
        for p in (0, 1):
            r = rr * 2 + p
            nxt = (p + 1) % 2

            @pl.when(r + 1 < ROUNDS)
            def _():
                start(r + 1, nxt)

            pltpu.make_async_copy(table_hbm.at[idxf_b[p]], rows_b[p],
                                  sems[p]).wait()
            process(r, p)
        return carry

    lax.fori_loop(0, ROUNDS // 2, outer, 0)


@jax.jit
def _pooled_lookup(emb_table_t, tail_lin):
    mesh = plsc.VectorSubcoreMesh(core_axis_name="c", subcore_axis_name="s")
    untile = functools.partial(
        pl.kernel,
        mesh=mesh,
        out_type=jax.ShapeDtypeStruct((VOCAB * DIM,), jnp.float32),
        scratch_types=[
            pltpu.VMEM((DIM, VC), jnp.float32),
            pltpu.VMEM((DIM, VC), jnp.float32),
            pltpu.VMEM((VC * DIM,), jnp.float32),
            pltpu.SemaphoreType.DMA,
            pltpu.SemaphoreType.DMA,
        ],
        compiler_params=pltpu.CompilerParams(use_tc_tiling_on_sc=True,
                                             needs_layout_passes=False),
    )(_untile_kernel)
    return untile(emb_table_t, tail_lin)


@jax.jit
def _lookup(table_lin, idx_t):
    mesh = plsc.VectorSubcoreMesh(core_axis_name="c", subcore_axis_name="s")
    f = functools.partial(
        pl.kernel,
        mesh=mesh,
        out_type=jax.ShapeDtypeStruct((DIM, BATCH_), jnp.float32),
        scratch_types=[
            pltpu.VMEM((HIST, CHUNK), jnp.int32),
            pltpu.VMEM((HIST, CHUNK), jnp.int32),
            pltpu.VMEM((ROWS,), jnp.int32),
            pltpu.VMEM((ROWS,), jnp.int32),
            pltpu.VMEM((ROWS, DIM), jnp.float32),
            pltpu.VMEM((ROWS, DIM), jnp.float32),
            pltpu.VMEM((DIM, CHUNK), jnp.float32),
            pltpu.SemaphoreType.DMA,
            pltpu.SemaphoreType.DMA,
        ],
        compiler_params=pltpu.CompilerParams(use_tc_tiling_on_sc=False,
                                             needs_layout_passes=False),
    )(_sc_kernel)
    return f(table_lin, idx_t)


def kernel(emb_table, inputs):
    tail_lin = emb_table[VOCAB - 64:].reshape(64 * DIM)
    table_flat = _pooled_lookup(emb_table.T, tail_lin)
    table_lin = table_flat.reshape(VOCAB, DIM)
    out_t = _lookup(table_lin, inputs.T)
    return out_t.T


# speedup vs baseline: 1.9852x; 1.8556x over previous
import functools

import jax
import jax.numpy as jnp
from jax import lax
from jax.experimental import pallas as pl
from jax.experimental.pallas import tpu as pltpu
from jax.experimental.pallas import tpu_sc as plsc

VOCAB = 1000000
DIM = 32
BATCH_ = 16384
HIST = 50

NC = 2
NS = 16
NW = NC * NS
B_PER_W = BATCH_ // NW
CHUNK = 32
ROWS = CHUNK * HIST
ROUNDS = B_PER_W // CHUNK

VC = 1024
NCH = VOCAB // VC
TAIL0 = NCH * VC
TAILA = 512
TAILB = VOCAB - TAIL0 - TAILA
PAIRS = 15


def _untile_kernel(tt_hbm, tail_hbm, out_hbm, blk0, blk1, outblk, sem0, sem1):
    wid = lax.axis_index("s") * NC + lax.axis_index("c")
    iota16 = lax.iota(jnp.int32, 16)
    gvecs = []
    pvecs = []
    for k in range(16):
        rot = (iota16 + k) % 16
        gvecs.append(iota16 * VC + rot)
        pvecs.append(rot * DIM + iota16)
    blk_b = (blk0, blk1)
    sems = (sem0, sem1)

    def chunk_id(j):
        return wid + NW * j

    def start(j, p):
        v0 = chunk_id(j) * VC
        for d in range(DIM):
            pltpu.async_copy(tt_hbm.at[d, pl.ds(v0, VC)],
                             blk_b[p].at[pl.ds(d * VC, VC)], sems[p])

    def wait_in(j, p):
        v0 = chunk_id(j) * VC
        for d in range(DIM):
            pltpu.make_async_copy(tt_hbm.at[d, pl.ds(v0, VC)],
                                  blk_b[p].at[pl.ds(d * VC, VC)],
                                  sems[p]).wait()

    def transpose_block(blk, width):
        def g_body(g, c):
            vbase = g * 16
            for d0 in (0, 16):
                rbase = d0 * VC + vbase
                sbase = vbase * DIM + d0
                for k in range(16):
                    vec = plsc.load_gather(blk, [gvecs[k] + rbase])
                    plsc.store_scatter(outblk, [pvecs[k] + sbase], vec)
            return c

        lax.fori_loop(0, width // 16, g_body, 0)

    def process(j, p, width):
        transpose_block(blk_b[p], width)
        pltpu.sync_copy(outblk.at[pl.ds(0, width * DIM)],
                        out_hbm.at[pl.ds(chunk_id(j) * VC * DIM, width * DIM)])

    start(0, 0)

    def pair_body(jj, carry):
        for p in (0, 1):
            j = jj * 2 + p
            nxt = (p + 1) % 2

            @pl.when(jnp.logical_or(j + 1 < 2 * PAIRS,
                                    jnp.logical_and(j + 1 == 2 * PAIRS,
                                                    wid < 16)))
            def _():
                start(j + 1, nxt)

            wait_in(j, p)
            process(j, p, VC)
        return carry

    lax.fori_loop(0, PAIRS, pair_body, 0)

    @pl.when(wid < 16)
    def _():
        j = 2 * PAIRS
        wait_in(j, 0)
        process(j, 0, VC)

    @pl.when(wid == 16)
    def _():
        for d in range(DIM):
            pltpu.sync_copy(tt_hbm.at[d, pl.ds(TAIL0, TAILA)],
                            blk_b[1].at[pl.ds(d * VC, TAILA)])
        transpose_block(blk_b[1], TAILA)
        pltpu.sync_copy(outblk.at[pl.ds(0, TAILA * DIM)],
                        out_hbm.at[pl.ds(TAIL0 * DIM, TAILA * DIM)])

    @pl.when(wid == 17)
    def _():
        pltpu.sync_copy(tail_hbm, outblk.at[pl.ds(0, TAILB * DIM)])
        pltpu.sync_copy(outblk.at[pl.ds(0, TAILB * DIM)],
                        out_hbm.at[pl.ds((TAIL0 + TAILA) * DIM, TAILB * DIM)])


def _sc_kernel(table_hbm, idxt_hbm, outt_hbm,
               idx2d0, idx2d1, idxf0, idxf1, rows0, rows1, outt_v,
               sem0, sem1):
    wid = lax.axis_index("s") * NC + lax.axis_index("c")
    base_b0 = wid * B_PER_W
    zero = jnp.zeros((16,), jnp.float32)
    inv = jnp.float32(1.0 / HIST)
    iota16 = lax.iota(jnp.int32, 16)
    idx2d_b = (idx2d0, idx2d1)
    idxf_b = (idxf0, idxf1)
    rows_b = (rows0, rows1)
    sems = (sem0, sem1)

    def start(r, p):
        idx2d = idx2d_b[p]
        idxf = idxf_b[p]
        pltpu.sync_copy(idxt_hbm.at[:, pl.ds(base_b0 + r * CHUNK, CHUNK)],
                        idx2d)

        def flat_body(h, c):
            base = h * CHUNK
            idxf[pl.ds(base, 16)] = idx2d[h, pl.ds(0, 16)]
            idxf[pl.ds(base + 16, 16)] = idx2d[h, pl.ds(16, 16)]
            return c

        lax.fori_loop(0, HIST, flat_body, 0)
        pltpu.async_copy(table_hbm.at[idxf], rows_b[p], sems[p])

    def process(r, p):
        rows_v = rows_b[p]

        def batch_body(b, c):
            a0 = zero
            a1 = zero
            c0 = zero
            c1 = zero
            for h in range(0, HIST, 2):
                a0 = a0 + rows_v[h * CHUNK + b, pl.ds(0, 16)]
                a1 = a1 + rows_v[h * CHUNK + b, pl.ds(16, 16)]
                c0 = c0 + rows_v[(h + 1) * CHUNK + b, pl.ds(0, 16)]
                c1 = c1 + rows_v[(h + 1) * CHUNK + b, pl.ds(16, 16)]
            bvec = jnp.full((16,), b, jnp.int32)
            plsc.store_scatter(outt_v, [iota16, bvec], (a0 + c0) * inv)
            plsc.store_scatter(outt_v, [iota16 + 16, bvec], (a1 + c1) * inv)
            return c

        lax.fori_loop(0, CHUNK, batch_body, 0)
        pltpu.sync_copy(outt_v,
                        outt_hbm.at[:, pl.ds(base_b0 + r * CHUNK, CHUNK)])

    start(0, 0)

    def outer(rr, carry):
        for p in (0, 1):
            r = rr * 2 + p
            nxt = (p + 1) % 2

            @pl.when(r + 1 < ROUNDS)
            def _():
                start(r + 1, nxt)

            pltpu.make_async_copy(table_hbm.at[idxf_b[p]], rows_b[p],
                                  sems[p]).wait()
            process(r, p)
        return carry

    lax.fori_loop(0, ROUNDS // 2, outer, 0)


@jax.jit
def _pooled_lookup(emb_table_t, tail_lin):
    mesh = plsc.VectorSubcoreMesh(core_axis_name="c", subcore_axis_name="s")
    untile = functools.partial(
        pl.kernel,
        mesh=mesh,
        out_type=jax.ShapeDtypeStruct((VOCAB * DIM,), jnp.float32),
        scratch_types=[
            pltpu.VMEM((DIM * VC,), jnp.float32),
            pltpu.VMEM((DIM * VC,), jnp.float32),
            pltpu.VMEM((VC * DIM,), jnp.float32),
            pltpu.SemaphoreType.DMA,
            pltpu.SemaphoreType.DMA,
        ],
        compiler_params=pltpu.CompilerParams(use_tc_tiling_on_sc=True,
                                             needs_layout_passes=False),
    )(_untile_kernel)
    return untile(emb_table_t, tail_lin)


@jax.jit
def _lookup(table_lin, idx_t):
    mesh = plsc.VectorSubcoreMesh(core_axis_name="c", subcore_axis_name="s")
    f = functools.partial(
        pl.kernel,
        mesh=mesh,
        out_type=jax.ShapeDtypeStruct((DIM, BATCH_), jnp.float32),
        scratch_types=[
            pltpu.VMEM((HIST, CHUNK), jnp.int32),
            pltpu.VMEM((HIST, CHUNK), jnp.int32),
            pltpu.VMEM((ROWS,), jnp.int32),
            pltpu.VMEM((ROWS,), jnp.int32),
            pltpu.VMEM((ROWS, DIM), jnp.float32),
            pltpu.VMEM((ROWS, DIM), jnp.float32),
            pltpu.VMEM((DIM, CHUNK), jnp.float32),
            pltpu.SemaphoreType.DMA,
            pltpu.SemaphoreType.DMA,
        ],
        compiler_params=pltpu.CompilerParams(use_tc_tiling_on_sc=False,
                                             needs_layout_passes=False),
    )(_sc_kernel)
    return f(table_lin, idx_t)


def kernel(emb_table, inputs):
    tail_lin = emb_table[VOCAB - 64:].reshape(64 * DIM)
    table_flat = _pooled_lookup(emb_table.T, tail_lin)
    table_lin = table_flat.reshape(VOCAB, DIM)
    out_t = _lookup(table_lin, inputs.T)
    return out_t.T


# speedup vs baseline: 2.2835x; 1.1502x over previous
import functools

import jax
import jax.numpy as jnp
from jax import lax
from jax.experimental import pallas as pl
from jax.experimental.pallas import tpu as pltpu
from jax.experimental.pallas import tpu_sc as plsc

VOCAB = 1000000
DIM = 32
BATCH_ = 16384
HIST = 50

NC = 2
NS = 16
NW = NC * NS
B_PER_W = BATCH_ // NW
CHUNK = 32
ROWS = CHUNK * HIST
ROUNDS = B_PER_W // CHUNK

VC = 896
NCH = VOCAB // VC
TAILB = VOCAB - NCH * VC
PAIRS = 17


def _untile_kernel(tt_hbm, tail_hbm, out_hbm, blk0, blk1, outblk0, outblk1,
                   sem0, sem1, osem0, osem1):
    wid = lax.axis_index("s") * NC + lax.axis_index("c")
    iota16 = lax.iota(jnp.int32, 16)
    gvecs = []
    pvecs = []
    for k in range(16):
        rot = (iota16 + k) % 16
        gvecs.append(iota16 * VC + rot)
        pvecs.append(rot * DIM + iota16)
    blk_b = (blk0, blk1)
    sems = (sem0, sem1)
    outblk_b = (outblk0, outblk1)
    osems = (osem0, osem1)

    def chunk_id(j):
        return jnp.minimum(wid + NW * j, NCH - 1)

    def start(j, p):
        v0 = chunk_id(j) * VC
        for d in range(DIM):
            pltpu.async_copy(tt_hbm.at[d, pl.ds(v0, VC)],
                             blk_b[p].at[pl.ds(d * VC, VC)], sems[p])

    def wait_in(j, p):
        v0 = chunk_id(j) * VC
        for d in range(DIM):
            pltpu.make_async_copy(tt_hbm.at[d, pl.ds(v0, VC)],
                                  blk_b[p].at[pl.ds(d * VC, VC)],
                                  sems[p]).wait()

    def transpose_block(blk, outblk, width):
        def g_body(g, c):
            vbase = g * 16
            for d0 in (0, 16):
                rbase = d0 * VC + vbase
                sbase = vbase * DIM + d0
                for k in range(16):
                    vec = plsc.load_gather(blk, [gvecs[k] + rbase])
                    plsc.store_scatter(outblk, [pvecs[k] + sbase], vec)
            return c

        lax.fori_loop(0, width // 16, g_body, 0)

    def out_copy(j, q):
        return pltpu.make_async_copy(
            outblk_b[q],
            out_hbm.at[pl.ds(chunk_id(j) * VC * DIM, VC * DIM)], osems[q])

    def process(j, p, q):
        @pl.when(j >= 2)
        def _():
            out_copy(j - 2, q).wait()

        transpose_block(blk_b[p], outblk_b[q], VC)
        pltpu.async_copy(
            outblk_b[q],
            out_hbm.at[pl.ds(chunk_id(j) * VC * DIM, VC * DIM)], osems[q])

    start(0, 0)

    def pair_body(jj, carry):
        for p in (0, 1):
            j = jj * 2 + p
            nxt = (p + 1) % 2

            @pl.when(jnp.logical_or(j + 1 < 2 * PAIRS,
                                    jnp.logical_and(j + 1 == 2 * PAIRS,
                                                    wid < 28)))
            def _():
                start(j + 1, nxt)

            wait_in(j, p)
            process(j, p, p)
        return carry

    lax.fori_loop(0, PAIRS, pair_body, 0)

    @pl.when(wid < 28)
    def _():
        j = 2 * PAIRS
        wait_in(j, 0)
        process(j, 0, 0)
        out_copy(j, 0).wait()
        out_copy(2 * PAIRS - 1, 1).wait()

    @pl.when(wid >= 28)
    def _():
        out_copy(2 * PAIRS - 2, 0).wait()
        out_copy(2 * PAIRS - 1, 1).wait()

    @pl.when(wid == 17)
    def _():
        pltpu.sync_copy(tail_hbm, blk_b[0].at[pl.ds(0, TAILB * DIM)])
        pltpu.sync_copy(blk_b[0].at[pl.ds(0, TAILB * DIM)],
                        out_hbm.at[pl.ds(NCH * VC * DIM, TAILB * DIM)])


def _sc_kernel(table_hbm, idxt_hbm, outt_hbm,
               idx2d0, idx2d1, idxf0, idxf1, rows0, rows1, outt_v,
               sem0, sem1):
    wid = lax.axis_index("s") * NC + lax.axis_index("c")
    base_b0 = wid * B_PER_W
    zero = jnp.zeros((16,), jnp.float32)
    inv = jnp.float32(1.0 / HIST)
    iota16 = lax.iota(jnp.int32, 16)
    idx2d_b = (idx2d0, idx2d1)
    idxf_b = (idxf0, idxf1)
    rows_b = (rows0, rows1)
    sems = (sem0, sem1)

    def start(r, p):
        idx2d = idx2d_b[p]
        idxf = idxf_b[p]
        pltpu.sync_copy(idxt_hbm.at[:, pl.ds(base_b0 + r * CHUNK, CHUNK)],
                        idx2d)

        def flat_body(h, c):
            base = h * CHUNK
            idxf[pl.ds(base, 16)] = idx2d[h, pl.ds(0, 16)]
            idxf[pl.ds(base + 16, 16)] = idx2d[h, pl.ds(16, 16)]
            return c

        lax.fori_loop(0, HIST, flat_body, 0)
        pltpu.async_copy(table_hbm.at[idxf], rows_b[p], sems[p])

    def process(r, p):
        rows_v = rows_b[p]

        def batch_body(b, c):
            a0 = zero
            a1 = zero
            c0 = zero
            c1 = zero
            for h in range(0, HIST, 2):
                a0 = a0 + rows_v[h * CHUNK + b, pl.ds(0, 16)]
                a1 = a1 + rows_v[h * CHUNK + b, pl.ds(16, 16)]
                c0 = c0 + rows_v[(h + 1) * CHUNK + b, pl.ds(0, 16)]
                c1 = c1 + rows_v[(h + 1) * CHUNK + b, pl.ds(16, 16)]
            bvec = jnp.full((16,), b, jnp.int32)
            plsc.store_scatter(outt_v, [iota16, bvec], (a0 + c0) * inv)
            plsc.store_scatter(outt_v, [iota16 + 16, bvec], (a1 + c1) * inv)
            return c

        lax.fori_loop(0, CHUNK, batch_body, 0)
        pltpu.sync_copy(outt_v,
                        outt_hbm.at[:, pl.ds(base_b0 + r * CHUNK, CHUNK)])

    start(0, 0)

    def outer(rr, carry):
        for p in (0, 1):
            r = rr * 2 + p
            nxt = (p + 1) % 2

            @pl.when(r + 1 < ROUNDS)
            def _():
                start(r + 1, nxt)

            pltpu.make_async_copy(table_hbm.at[idxf_b[p]], rows_b[p],
                                  sems[p]).wait()
            process(r, p)
        return carry

    lax.fori_loop(0, ROUNDS // 2, outer, 0)


@jax.jit
def _pooled_lookup(emb_table_t, tail_lin):
    mesh = plsc.VectorSubcoreMesh(core_axis_name="c", subcore_axis_name="s")
    untile = functools.partial(
        pl.kernel,
        mesh=mesh,
        out_type=jax.ShapeDtypeStruct((VOCAB * DIM,), jnp.float32),
        scratch_types=[
            pltpu.VMEM((DIM * VC,), jnp.float32),
            pltpu.VMEM((DIM * VC,), jnp.float32),
            pltpu.VMEM((VC * DIM,), jnp.float32),
            pltpu.VMEM((VC * DIM,), jnp.float32),
            pltpu.SemaphoreType.DMA,
            pltpu.SemaphoreType.DMA,
            pltpu.SemaphoreType.DMA,
            pltpu.SemaphoreType.DMA,
        ],
        compiler_params=pltpu.CompilerParams(use_tc_tiling_on_sc=True,
                                             needs_layout_passes=False),
    )(_untile_kernel)
    return untile(emb_table_t, tail_lin)


@jax.jit
def _lookup(table_lin, idx_t):
    mesh = plsc.VectorSubcoreMesh(core_axis_name="c", subcore_axis_name="s")
    f = functools.partial(
        pl.kernel,
        mesh=mesh,
        out_type=jax.ShapeDtypeStruct((DIM, BATCH_), jnp.float32),
        scratch_types=[
            pltpu.VMEM((HIST, CHUNK), jnp.int32),
            pltpu.VMEM((HIST, CHUNK), jnp.int32),
            pltpu.VMEM((ROWS,), jnp.int32),
            pltpu.VMEM((ROWS,), jnp.int32),
            pltpu.VMEM((ROWS, DIM), jnp.float32),
            pltpu.VMEM((ROWS, DIM), jnp.float32),
            pltpu.VMEM((DIM, CHUNK), jnp.float32),
            pltpu.SemaphoreType.DMA,
            pltpu.SemaphoreType.DMA,
        ],
        compiler_params=pltpu.CompilerParams(use_tc_tiling_on_sc=False,
                                             needs_layout_passes=False),
    )(_sc_kernel)
    return f(table_lin, idx_t)


def kernel(emb_table, inputs):
    tail_lin = emb_table[VOCAB - 64:].reshape(64 * DIM)
    table_flat = _pooled_lookup(emb_table.T, tail_lin)
    table_lin = table_flat.reshape(VOCAB, DIM)
    out_t = _lookup(table_lin, inputs.T)
    return out_t.T
